# baseline (speedup 1.0000x reference)
import functools

import jax
import jax.numpy as jnp
from jax import lax
from jax.experimental import pallas as pl
from jax.experimental.pallas import tpu as pltpu
from jax.experimental.pallas import tpu_sc as plsc

D = 64
MARGIN = 1.0
LANES = 16


@functools.cache
def _make_kernel(B: int):
    NC, NS = 2, 16
    NW = NC * NS
    W = B // NW
    CH = 64
    NCHUNK = W // CH

    mesh = plsc.VectorSubcoreMesh(core_axis_name="c", subcore_axis_name="s")

    @functools.partial(
        pl.kernel,
        mesh=mesh,
        compiler_params=pltpu.CompilerParams(
            needs_layout_passes=False, use_tc_tiling_on_sc=False),
        out_type=jax.ShapeDtypeStruct((NW, LANES), jnp.float32),
        scratch_types=[
            pltpu.VMEM((6, W), jnp.int32),
            pltpu.VMEM((12, CH, D), jnp.float32),
            pltpu.VMEM((LANES, LANES), jnp.float32),
            pltpu.VMEM((LANES,), jnp.float32),
            pltpu.SemaphoreType.DMA,
        ],
    )
    def k(ph, pr, pt, nh, nr, nt, ent_r, ent_i, rel_r, rel_i,
          out, idx_v, rows_v, part_v, res_v, sem):
        wid = lax.axis_index("s") * NC + lax.axis_index("c")
        base = wid * W

        for j, src in enumerate((ph, pr, pt, nh, nr, nt)):
            pltpu.sync_copy(src.at[pl.ds(base, W)], idx_v.at[j])

        plan = ((ent_r, 0), (ent_i, 0), (rel_r, 1), (rel_i, 1),
                (ent_r, 2), (ent_i, 2),
                (ent_r, 3), (ent_i, 3), (rel_r, 4), (rel_i, 4),
                (ent_r, 5), (ent_i, 5))

        def score(i, h_slot, r_slot, t_slot):
            p = jnp.zeros((LANES,), jnp.float32)
            for kk in range(D // LANES):
                sl = pl.ds(kk * LANES, LANES)
                hr = rows_v[h_slot, i, sl]
                hi = rows_v[h_slot + 1, i, sl]
                rr = rows_v[r_slot, i, sl]
                ri = rows_v[r_slot + 1, i, sl]
                tr = rows_v[t_slot, i, sl]
                ti = rows_v[t_slot + 1, i, sl]
                p = p + rr * (hr * tr + hi * ti) + ri * (hr * ti - hi * tr)
            return p

        lane = lax.iota(jnp.int32, LANES)

        def pair_body(ii, g):
            i = g * LANES + ii
            part_v[ii, :] = score(i, 6, 8, 10) - score(i, 0, 2, 4)
            return g

        def group_body(g, acc):
            lax.fori_loop(0, LANES, pair_body, g)
            s = jnp.zeros((LANES,), jnp.float32)
            for j in range(LANES):
                s = s + plsc.load_gather(
                    part_v, [lane, jnp.full((LANES,), j, jnp.int32)])
            return acc + jnp.maximum(s + MARGIN, 0.0)

        def chunk_body(c, acc):
            copies = [
                pltpu.async_copy(
                    tbl.at[idx_v.at[jslot, pl.ds(c * CH, CH)]],
                    rows_v.at[slot], sem)
                for slot, (tbl, jslot) in enumerate(plan)
            ]
            for cp in copies:
                cp.wait()
            return lax.fori_loop(0, CH // LANES, group_body, acc)

        acc = lax.fori_loop(0, NCHUNK, chunk_body,
                            jnp.zeros((LANES,), jnp.float32))

        res_v[...] = acc
        pltpu.sync_copy(res_v, out.at[wid])

    return k


def kernel(pos_exmpl, neg_exmpl, ent_real, ent_imag, rel_real, rel_imag):
    B = pos_exmpl.shape[1]
    k = _make_kernel(B)
    out = k(pos_exmpl[0], pos_exmpl[1], pos_exmpl[2],
            neg_exmpl[0], neg_exmpl[1], neg_exmpl[2],
            ent_real, ent_imag, rel_real, rel_imag)
    return jnp.sum(out)
